# baseline (device time: 306468 ns/iter reference)
import jax
import jax.numpy as jnp
from jax import lax
from jax.experimental import pallas as pl
from jax.experimental.pallas import tpu as pltpu


def kernel(Q, K, V):
    b, q, h, d = Q.shape
    kloc = K.shape[1]
    hd = h * d
    scale = d ** -0.5

    Q3 = Q.reshape(b, q, hd)
    K3 = K.reshape(b, kloc, hd)
    V3 = V.reshape(b, kloc, hd)

    n_chunks = 4
    kc = kloc // n_chunks

    def partial_body(q_ref, k_ref, v_ref, n_ref, d_ref):
        c = pl.program_id(1)
        for hh in range(h):
            sl = slice(hh * d, (hh + 1) * d)
            qh = q_ref[0][:, sl]
            kh = k_ref[0][:, sl]
            vh = v_ref[0][:, sl]
            s = lax.dot_general(
                qh, kh, (((1,), (1,)), ((), ())),
                preferred_element_type=jnp.float32) * scale
            p = jnp.exp(s)
            n = lax.dot_general(
                p, vh, (((1,), (0,)), ((), ())),
                preferred_element_type=jnp.float32)
            l = jnp.broadcast_to(
                jnp.sum(p, axis=1, keepdims=True), (q, d))

            @pl.when(c == 0)
            def _():
                n_ref[0, :, sl] = n
                d_ref[0, :, sl] = l

            @pl.when(c != 0)
            def _():
                n_ref[0, :, sl] += n
                d_ref[0, :, sl] += l

    N3, D3 = pl.pallas_call(
        partial_body,
        grid=(b, n_chunks),
        in_specs=[
            pl.BlockSpec((1, q, hd), lambda i, c: (i, 0, 0)),
            pl.BlockSpec((1, kc, hd), lambda i, c: (i, c, 0)),
            pl.BlockSpec((1, kc, hd), lambda i, c: (i, c, 0)),
        ],
        out_specs=[
            pl.BlockSpec((1, q, hd), lambda i, c: (i, 0, 0)),
            pl.BlockSpec((1, q, hd), lambda i, c: (i, 0, 0)),
        ],
        out_shape=[
            jax.ShapeDtypeStruct((b, q, hd), jnp.float32),
            jax.ShapeDtypeStruct((b, q, hd), jnp.float32),
        ],
    )(Q3, K3, V3)

    def reduce_body(n_ref, d_ref, o_ref, ncom_ref, dcom_ref,
                    send_sem, recv_sem):
        my_x = lax.axis_index("x")
        my_y = lax.axis_index("y")
        my_z = lax.axis_index("z")
        nbr = (1 - my_x, my_y, my_z)
        copy_n = pltpu.make_async_remote_copy(
            src_ref=n_ref, dst_ref=ncom_ref,
            send_sem=send_sem.at[0], recv_sem=recv_sem.at[0],
            device_id=nbr, device_id_type=pl.DeviceIdType.MESH)
        copy_d = pltpu.make_async_remote_copy(
            src_ref=d_ref, dst_ref=dcom_ref,
            send_sem=send_sem.at[1], recv_sem=recv_sem.at[1],
            device_id=nbr, device_id_type=pl.DeviceIdType.MESH)
        copy_n.start()
        copy_d.start()
        copy_n.wait()
        copy_d.wait()
        o_ref[...] = (n_ref[...] + ncom_ref[...]) / (d_ref[...] + dcom_ref[...])

    O3 = pl.pallas_call(
        reduce_body,
        in_specs=[pl.BlockSpec(memory_space=pltpu.VMEM)] * 2,
        out_specs=pl.BlockSpec(memory_space=pltpu.VMEM),
        out_shape=jax.ShapeDtypeStruct((b, q, hd), jnp.float32),
        scratch_shapes=[
            pltpu.VMEM((b, q, hd), jnp.float32),
            pltpu.VMEM((b, q, hd), jnp.float32),
            pltpu.SemaphoreType.DMA((2,)),
            pltpu.SemaphoreType.DMA((2,)),
        ],
    )(N3, D3)

    return O3.reshape(b, q, h, d)
